# Initial kernel scaffold; baseline (speedup 1.0000x reference)
import jax
import jax.numpy as jnp
from jax.experimental import pallas as pl


def kernel(q, v, embeddings):
    raise NotImplementedError("write your pallas kernel here")


# speedup vs baseline: 5.8269x; 5.8269x over previous
import jax
import jax.numpy as jnp
from jax import lax
from jax.experimental import pallas as pl
from jax.experimental.pallas import tpu as pltpu
from jax.experimental.pallas import tpu_sc as plsc

_Q_LEN = 2048
_V_LEN = 2048
_OUT_DIM = 12
_ROW = _V_LEN * _OUT_DIM
_G_LEN = 49152
_NC = 2
_NS = 16
_ROWS_PER_TILE = _Q_LEN // (_NC * _NS)
_CHUNK = _G_LEN // _NS
_ITERS = _CHUNK // 16


def _body(emb_hbm, out_hbm, emb_ts, buf0, buf4, g0, g4, sem):
    c = lax.axis_index("c")
    s = lax.axis_index("s")

    pltpu.sync_copy(emb_hbm, emb_ts)

    lanes = lax.iota(jnp.int32, 16)

    def make_fill(buf, shift):
        def fill(j, carry):
            p = _CHUNK * s + 16 * j + lanes + shift
            d = lax.div(p, _OUT_DIM)
            k = p - d * _OUT_DIM
            n = jnp.abs(d - (_Q_LEN - 1))
            side = jnp.where(d > (_Q_LEN - 1), 16, 0)
            nsq_f = (n * n).astype(jnp.float32)
            e = lax.bitcast_convert_type(nsq_f, jnp.int32) >> 23
            val_large = jnp.minimum(e - 125, 15)
            bucket = side + jnp.where(n < 8, n, val_large)
            buf[pl.ds(16 * j, 16)] = plsc.load_gather(emb_ts, [bucket * _OUT_DIM + k])
            return carry

        return fill

    lax.fori_loop(0, _ITERS, make_fill(buf0, 0), 0)
    lax.fori_loop(0, _ITERS, make_fill(buf4, 4), 0)
    pltpu.sync_copy(buf0, g0.at[pl.ds(_CHUNK * s, _CHUNK)])
    pltpu.sync_copy(buf4, g4.at[pl.ds(_CHUNK * s, _CHUNK)])
    plsc.subcore_barrier()

    base = (s * _NC + c) * _ROWS_PER_TILE

    def fire(t, carry):
        i_even = base + 2 * t
        o_even = _OUT_DIM * ((_Q_LEN - 1) - i_even)
        pltpu.async_copy(
            g4.at[pl.ds(pl.multiple_of(o_even - 4, 8), _ROW)],
            out_hbm.at[pl.ds(i_even * _ROW, _ROW)],
            sem,
        )
        i_odd = i_even + 1
        o_odd = _OUT_DIM * ((_Q_LEN - 1) - i_odd)
        pltpu.async_copy(
            g0.at[pl.ds(pl.multiple_of(o_odd, 8), _ROW)],
            out_hbm.at[pl.ds(i_odd * _ROW, _ROW)],
            sem,
        )
        return carry

    lax.fori_loop(0, _ROWS_PER_TILE // 2, fire, 0)

    def drain(t, carry):
        pltpu.make_async_copy(
            g0.at[pl.ds(0, _ROW)], out_hbm.at[pl.ds(0, _ROW)], sem
        ).wait()
        return carry

    lax.fori_loop(0, _ROWS_PER_TILE, drain, 0)


_sc_expand = pl.kernel(
    _body,
    out_type=jax.ShapeDtypeStruct((_Q_LEN * _ROW,), jnp.float32),
    mesh=plsc.VectorSubcoreMesh(
        core_axis_name="c", subcore_axis_name="s", num_cores=_NC, num_subcores=_NS
    ),
    scratch_types=[
        pltpu.VMEM((384,), jnp.float32),
        pltpu.VMEM((_CHUNK,), jnp.float32),
        pltpu.VMEM((_CHUNK,), jnp.float32),
        pltpu.VMEM_SHARED((_G_LEN,), jnp.float32),
        pltpu.VMEM_SHARED((_G_LEN,), jnp.float32),
        pltpu.SemaphoreType.DMA,
    ],
    compiler_params=pltpu.CompilerParams(
        needs_layout_passes=False, use_tc_tiling_on_sc=False
    ),
)


@jax.jit
def kernel(q, v, embeddings):
    del q, v
    out = _sc_expand(embeddings.reshape(-1))
    return out.reshape(_Q_LEN, _V_LEN, _OUT_DIM)


# speedup vs baseline: 68.3541x; 11.7308x over previous
import jax
import jax.numpy as jnp
from jax import lax
from jax.experimental import pallas as pl
from jax.experimental.pallas import tpu as pltpu
from jax.experimental.pallas import tpu_sc as plsc

_Q_LEN = 2048
_OUT_DIM = 12
_NC = 2
_NS = 16
_GK_PAD = 4160
_JJ = 496
_MR_ROWS = _JJ * 8
_JPT = _JJ // _NS
_RPT = _JPT * 8
_FPT = 128


def _body(emb_hbm, out_hbm, emb_ts, gk, mbuf, mr, sem):
    c = lax.axis_index("c")
    s = lax.axis_index("s")
    wid = s * _NC + c
    lanes = lax.iota(jnp.int32, 16)

    pltpu.sync_copy(emb_hbm, emb_ts)

    def drain(t, carry):
        pltpu.make_async_copy(
            mr.at[0, pl.ds(0, 8), :],
            out_hbm.at[0, pl.ds(0, 8), pl.ds(0, 128)],
            sem,
        ).wait()
        return carry

    for r in range(_OUT_DIM):
        k = r
        slot = r % 3
        if r >= 2:
            lax.fori_loop(0, _FPT, drain, 0)
        plsc.subcore_barrier()

        def fill_gk(jd, carry, _k=k):
            q = 16 * jd + lanes
            n = jnp.abs(q - (_Q_LEN - 1))
            side = jnp.where(q > (_Q_LEN - 1), 16, 0)
            nsq_f = (n * n).astype(jnp.float32)
            e = lax.bitcast_convert_type(nsq_f, jnp.int32) >> 23
            val_large = jnp.minimum(e - 125, 15)
            bucket = side + jnp.where(n < 8, n, val_large)
            gk[pl.ds(16 * jd, 16)] = plsc.load_gather(
                emb_ts, [bucket * _OUT_DIM + _k]
            )
            return carry

        lax.fori_loop(0, 4096 // 16, fill_gk, 0)

        def build_rows(row0, nrows, buf_rows):
            def br(rr, carry):
                row = row0 + rr

                def bc(cc, carry2):
                    idx = (3967 - row + 16 * cc) + lanes
                    mbuf[rr, pl.ds(16 * cc, 16)] = plsc.load_gather(gk, [idx])
                    return carry2

                return lax.fori_loop(0, 8, bc, carry)

            lax.fori_loop(0, nrows, br, 0)
            pltpu.sync_copy(
                mbuf.at[pl.ds(0, buf_rows), :],
                mr.at[slot, pl.ds(row0, buf_rows), :],
            )

        build_rows(_RPT * s, 128, 128)
        build_rows(_RPT * s + 128, 120, 120)
        plsc.subcore_barrier()

        def fire(ff, carry, _k=k, _slot=slot):
            st = wid * _FPT + ff
            ti = st >> 4
            tj = st & 15
            jj = 240 - 16 * tj + ti
            pltpu.async_copy(
                mr.at[_slot, pl.ds(pl.multiple_of(8 * jj, 8), 8), :],
                out_hbm.at[
                    _k,
                    pl.ds(pl.multiple_of(8 * ti, 8), 8),
                    pl.ds(pl.multiple_of(128 * tj, 128), 128),
                ],
                sem,
            )
            return carry

        lax.fori_loop(0, _FPT, fire, 0)

    lax.fori_loop(0, 2 * _FPT, drain, 0)


_sc_expand = pl.kernel(
    _body,
    out_type=jax.ShapeDtypeStruct((_OUT_DIM, _Q_LEN, _Q_LEN), jnp.float32),
    mesh=plsc.VectorSubcoreMesh(
        core_axis_name="c", subcore_axis_name="s", num_cores=_NC, num_subcores=_NS
    ),
    scratch_types=[
        pltpu.VMEM((384,), jnp.float32),
        pltpu.VMEM((_GK_PAD,), jnp.float32),
        pltpu.VMEM((128, 128), jnp.float32),
        pltpu.VMEM_SHARED((3, _MR_ROWS, 128), jnp.float32),
        pltpu.SemaphoreType.DMA,
    ],
    compiler_params=pltpu.CompilerParams(
        needs_layout_passes=False, use_tc_tiling_on_sc=True
    ),
)


@jax.jit
def kernel(q, v, embeddings):
    del q, v
    out = _sc_expand(embeddings.reshape(-1))
    return jnp.transpose(out, (1, 2, 0))
